# baseline (speedup 1.0000x reference)
import jax
import jax.numpy as jnp
from jax import lax
from jax.experimental import pallas as pl
from jax.experimental.pallas import tpu as pltpu
from jax.experimental.pallas import tpu_sc as plsc

VOCAB = 1_000_000
BATCH = 16384
ROWS, COLS = 1000, 1000
NBLK = 5
BLK_ROWS = ROWS // NBLK

X_ROWS, X_COLS = 128, 128


def _lse_body(x_ref, o_ref, m_ref, s_ref):
    i = pl.program_id(0)
    blk = x_ref[...]
    bm = jnp.max(blk).reshape(1, 1)
    bs = jnp.sum(jnp.exp(blk - bm)).reshape(1, 1)

    @pl.when(i == 0)
    def _init():
        m_ref[...] = jnp.full((1, 1), -jnp.inf, jnp.float32)
        s_ref[...] = jnp.zeros((1, 1), jnp.float32)

    m_old = m_ref[...]
    m_new = jnp.maximum(m_old, bm)
    s_ref[...] = s_ref[...] * jnp.exp(m_old - m_new) + bs * jnp.exp(bm - m_new)
    m_ref[...] = m_new

    @pl.when(i == NBLK - 1)
    def _fin():
        o_ref[...] = m_ref[...] + jnp.log(s_ref[...])


def _logsumexp(logits2d):
    return pl.pallas_call(
        _lse_body,
        grid=(NBLK,),
        in_specs=[pl.BlockSpec((BLK_ROWS, COLS), lambda i: (i, 0))],
        out_specs=pl.BlockSpec((1, 1), lambda i: (0, 0)),
        out_shape=jax.ShapeDtypeStruct((1, 1), jnp.float32),
        scratch_shapes=[
            pltpu.VMEM((1, 1), jnp.float32),
            pltpu.VMEM((1, 1), jnp.float32),
        ],
    )(logits2d)


_INFO = plsc.get_sparse_core_info()
_NC = _INFO.num_cores
_NS = _INFO.num_subcores
_NW = _NC * _NS
_RPW = X_ROWS // _NW


def _gather_body(tab_ref, x_ref, lse_ref, o_ref, idx_v, rows_v, lse_v, sem):
    wid = lax.axis_index("s") * _NC + lax.axis_index("c")
    r0 = wid * _RPW
    pltpu.sync_copy(x_ref.at[pl.ds(r0, _RPW)], idx_v)
    pltpu.sync_copy(lse_ref, lse_v)
    for j in range(_RPW):
        pltpu.async_copy(tab_ref.at[idx_v.at[j]], rows_v.at[j], sem).wait()
    lse_vec = lse_v[...]
    for j in range(_RPW):
        for k in range(128 // 16):
            sl = pl.ds(k * 16, 16)
            rows_v[j, sl] = rows_v[j, sl] - lse_vec
    pltpu.sync_copy(rows_v, o_ref.at[pl.ds(r0, _RPW)])


def _gather_sub(logits, x2d, lse16):
    mesh = plsc.VectorSubcoreMesh(core_axis_name="c", subcore_axis_name="s")
    f = pl.kernel(
        _gather_body,
        mesh=mesh,
        out_type=jax.ShapeDtypeStruct((X_ROWS, X_COLS), jnp.float32),
        scratch_types=[
            pltpu.VMEM((_RPW, X_COLS), jnp.int32),
            pltpu.VMEM((_RPW, X_COLS), jnp.float32),
            pltpu.VMEM((16,), jnp.float32),
            pltpu.SemaphoreType.DMA,
        ],
    )
    return f(logits, x2d, lse16)


def kernel(logits, x):
    lse = _logsumexp(logits.reshape(ROWS, COLS))
    lse16 = jnp.broadcast_to(lse.reshape(1), (16,))
    x2d = x.reshape(X_ROWS, X_COLS)
    out = _gather_sub(logits, x2d, lse16)
    return out.reshape(BATCH)


# speedup vs baseline: 1.1342x; 1.1342x over previous
import jax
import jax.numpy as jnp
from jax import lax
from jax.experimental import pallas as pl
from jax.experimental.pallas import tpu as pltpu
from jax.experimental.pallas import tpu_sc as plsc

VOCAB = 1_000_000
BATCH = 16384
ROWS, COLS = 1000, 1000
NBLK = 5
BLK_ROWS = ROWS // NBLK

X_ROWS, X_COLS = 128, 128


def _lse_body(x_ref, o_ref, m_ref, s_ref):
    i = pl.program_id(0)
    blk = x_ref[...]
    bm = jnp.max(blk).reshape(1, 1)
    bs = jnp.sum(jnp.exp(blk - bm)).reshape(1, 1)

    @pl.when(i == 0)
    def _init():
        m_ref[...] = jnp.full((1, 1), -jnp.inf, jnp.float32)
        s_ref[...] = jnp.zeros((1, 1), jnp.float32)

    m_old = m_ref[...]
    m_new = jnp.maximum(m_old, bm)
    s_ref[...] = s_ref[...] * jnp.exp(m_old - m_new) + bs * jnp.exp(bm - m_new)
    m_ref[...] = m_new

    @pl.when(i == NBLK - 1)
    def _fin():
        o_ref[...] = m_ref[...] + jnp.log(s_ref[...])


def _logsumexp(logits2d):
    return pl.pallas_call(
        _lse_body,
        grid=(NBLK,),
        in_specs=[pl.BlockSpec((BLK_ROWS, COLS), lambda i: (i, 0))],
        out_specs=pl.BlockSpec((1, 1), lambda i: (0, 0)),
        out_shape=jax.ShapeDtypeStruct((1, 1), jnp.float32),
        scratch_shapes=[
            pltpu.VMEM((1, 1), jnp.float32),
            pltpu.VMEM((1, 1), jnp.float32),
        ],
    )(logits2d)


_INFO = plsc.get_sparse_core_info()
_NC = _INFO.num_cores
_NS = _INFO.num_subcores
_NW = _NC * _NS
_RPW = X_ROWS // _NW


def _gather_body(tab_ref, x_ref, o_ref, idx_v, rows_v, sem):
    wid = lax.axis_index("s") * _NC + lax.axis_index("c")
    r0 = wid * _RPW
    pltpu.sync_copy(x_ref.at[pl.ds(r0, _RPW)], idx_v)
    copies = [
        pltpu.async_copy(tab_ref.at[idx_v.at[j]], rows_v.at[j], sem)
        for j in range(_RPW)
    ]
    for c in copies:
        c.wait()
    pltpu.sync_copy(rows_v, o_ref.at[pl.ds(r0, _RPW)])


def _gather(logits, x2d):
    mesh = plsc.VectorSubcoreMesh(core_axis_name="c", subcore_axis_name="s")
    f = pl.kernel(
        _gather_body,
        mesh=mesh,
        out_type=jax.ShapeDtypeStruct((X_ROWS, X_COLS), jnp.float32),
        scratch_types=[
            pltpu.VMEM((_RPW, X_COLS), jnp.int32),
            pltpu.VMEM((_RPW, X_COLS), jnp.float32),
            pltpu.SemaphoreType.DMA,
        ],
    )
    return f(logits, x2d)


def kernel(logits, x):
    lse = _logsumexp(logits.reshape(ROWS, COLS))
    x2d = x.reshape(X_ROWS, X_COLS)
    g = _gather(logits, x2d)
    return (g - lse[0, 0]).reshape(BATCH)
